# baseline (speedup 1.0000x reference)
import functools
import jax
import jax.numpy as jnp
from jax.experimental import pallas as pl

_K = 256


def _score_body(w_ref, f_ref, s_ref):
    f = f_ref[0]
    w = w_ref[...]
    s_ref[0] = jnp.dot(w, f, preferred_element_type=jnp.float32)


def _select_body(s_ref, m_ref):
    s = s_ref[...]
    B, V = s.shape
    _INT_MIN = jnp.int32(-2147483648)
    ki = jax.lax.bitcast_convert_type(s, jnp.int32)
    key = jnp.where(ki >= 0, ki, ki ^ jnp.int32(0x7FFFFFFF))
    x = key ^ _INT_MIN

    prefix = jnp.zeros((B, 1), jnp.int32)
    need = jnp.full((B, 1), _K, jnp.int32)
    for bit in range(31, -1, -1):
        b = jnp.int32(1 << bit) if bit < 31 else _INT_MIN
        lowmask = jnp.int32((1 << (bit + 1)) - 1) if bit < 31 else jnp.int32(-1)
        himask = ~lowmask
        cand_hi = ((x & himask) == prefix) & ((x & b) != 0)
        c1 = jnp.sum(cand_hi.astype(jnp.int32), axis=1, keepdims=True)
        go_hi = c1 >= need
        prefix = jnp.where(go_hi, prefix | b, prefix)
        need = jnp.where(go_hi, need, need - c1)
    t_key = prefix ^ _INT_MIN
    gt = key > t_key
    eq = key == t_key

    idx = jax.lax.broadcasted_iota(jnp.int32, (B, V), 1)
    prefix2 = jnp.zeros((B, 1), jnp.int32)
    need2 = need
    for bit in range(14, -1, -1):
        b = jnp.int32(1 << bit)
        himask2 = ~jnp.int32((1 << (bit + 1)) - 1)
        cand_lo = eq & ((idx & himask2) == prefix2) & ((idx & b) == 0)
        c0 = jnp.sum(cand_lo.astype(jnp.int32), axis=1, keepdims=True)
        stay_lo = c0 >= need2
        prefix2 = jnp.where(stay_lo, prefix2, prefix2 | b)
        need2 = jnp.where(stay_lo, need2, need2 - c0)
    t2 = prefix2

    sel = gt | (eq & (idx <= t2))
    m_ref[...] = sel.astype(jnp.float32)


def _mean_body(f_ref, m_ref, o_ref, *, nj):
    j = pl.program_id(1)

    @pl.when(j == 0)
    def _():
        o_ref[...] = jnp.zeros_like(o_ref)

    f = f_ref[0]
    m = m_ref[0]
    o_ref[0] += jax.lax.dot_general(
        m, f, (((1,), (1,)), ((), ())),
        preferred_element_type=jnp.float32)

    @pl.when(j == nj - 1)
    def _():
        o_ref[0] *= jnp.float32(1.0 / _K)


def kernel(Fmap, score_w, score_b):
    B, C, D, H, W = Fmap.shape
    V = D * H * W
    f = Fmap.reshape(B, C, V)
    w = score_w.reshape(1, C)

    BV = 2048
    nj = V // BV

    s = pl.pallas_call(
        _score_body,
        grid=(B, nj),
        in_specs=[
            pl.BlockSpec((1, C), lambda b, j: (0, 0)),
            pl.BlockSpec((1, C, BV), lambda b, j: (b, 0, j)),
        ],
        out_specs=pl.BlockSpec((1, 1, BV), lambda b, j: (b * nj + j, 0, 0)),
        out_shape=jax.ShapeDtypeStruct((B * nj, 1, BV), jnp.float32),
    )(w, f)

    mask = pl.pallas_call(
        _select_body,
        out_shape=jax.ShapeDtypeStruct((B, V), jnp.float32),
    )(s.reshape(B, V))

    out = pl.pallas_call(
        functools.partial(_mean_body, nj=nj),
        grid=(B, nj),
        in_specs=[
            pl.BlockSpec((1, C, BV), lambda b, j: (b, 0, j)),
            pl.BlockSpec((1, 1, BV), lambda b, j: (b * nj + j, 0, 0)),
        ],
        out_specs=pl.BlockSpec((1, 1, C), lambda b, j: (b, 0, 0)),
        out_shape=jax.ShapeDtypeStruct((B, 1, C), jnp.float32),
    )(f, mask.reshape(B * nj, 1, BV))

    return out.reshape(B, C)
